# baseline (speedup 1.0000x reference)
import functools

import jax
import jax.numpy as jnp
from jax import lax
from jax.experimental import pallas as pl
from jax.experimental.pallas import tpu as pltpu
from jax.experimental.pallas import tpu_sc as plsc

NUM_K = 8192
DIM = 32
PIX = 256
NPIX = 2 * PIX
BETA = 0.25
K_CHUNK = 2048
N_CHUNKS = NUM_K // K_CHUNK

NW = 16
ROWS_W = NPIX // NW


def _dist_kernel(zs_ref, et_ref, idx_ref):
    for b in range(2):
        zs = zs_ref[b]

        def chunk_body(kc, carry):
            best_val, best_idx = carry
            base = kc * K_CHUNK
            acc = None
            for c in range(DIM):
                er = et_ref[c, pl.ds(base * 1, K_CHUNK)].reshape(1, K_CHUNK)
                zc = zs[:, c].reshape(PIX, 1)
                d = er - zc
                sq = d * d
                acc = sq if acc is None else acc + sq
            vmin = jnp.min(acc, axis=1, keepdims=True)
            kiota = jax.lax.broadcasted_iota(jnp.int32, (PIX, K_CHUNK), 1)
            ilocal = jnp.min(
                jnp.where(acc == vmin, kiota, NUM_K), axis=1, keepdims=True
            )
            cand_idx = ilocal + base
            better = vmin < best_val
            best_val = jnp.where(better, vmin, best_val)
            best_idx = jnp.where(better, cand_idx, best_idx)
            return best_val, best_idx

        init = (
            jnp.full((PIX, 1), jnp.inf, dtype=jnp.float32),
            jnp.zeros((PIX, 1), dtype=jnp.int32),
        )
        _, best_idx = jax.lax.fori_loop(0, N_CHUNKS, chunk_body, init)
        idx_ref[b] = best_idx


_SC_MESH = plsc.VectorSubcoreMesh(
    core_axis_name="c", subcore_axis_name="s", num_cores=1
)


@functools.partial(
    pl.kernel,
    mesh=_SC_MESH,
    out_type=jax.ShapeDtypeStruct((NPIX, 128), jnp.float32),
    scratch_types=[
        pltpu.VMEM((ROWS_W,), jnp.int32),
        pltpu.VMEM((ROWS_W, 128), jnp.float32),
        pltpu.SemaphoreType.DMA,
    ],
)
def _gather_kernel(emb_hbm, idx_hbm, rows_hbm, idx_v, rows_v, sem):
    w = lax.axis_index("s")
    base = w * ROWS_W
    pltpu.sync_copy(idx_hbm.at[pl.ds(base, ROWS_W)], idx_v)
    pltpu.async_copy(emb_hbm.at[idx_v], rows_v, sem).wait()
    pltpu.sync_copy(rows_v, rows_hbm.at[pl.ds(base, ROWS_W)])


def _epilogue_kernel(rows_ref, zn_ref, zqt_ref, loss_ref):
    loss_acc = jnp.zeros((), dtype=jnp.float32)
    for b in range(2):
        dn = rows_ref[b][:, :DIM] - zn_ref[b]
        st = zn_ref[b] + dn
        zqt_ref[b] = st.T
        loss_acc = loss_acc + jnp.sum(dn * dn)
    scale = (1.0 + BETA) / (NPIX * DIM)
    loss_ref[...] = (loss_acc * scale).reshape(1, 1)


def kernel(z, embedding):
    b, c, h, w = z.shape
    zp = jnp.transpose(z, (0, 2, 3, 1))
    flat = zp.reshape(b, h * w * c)
    zs = flat.reshape(b, c, h * w).transpose(0, 2, 1)
    zn = zp.reshape(b, h * w, c)
    et = embedding.T

    idx = pl.pallas_call(
        _dist_kernel,
        out_shape=jax.ShapeDtypeStruct((b, h * w, 1), jnp.int32),
    )(zs, et)

    emb_pad = jnp.pad(embedding, ((0, 0), (0, 128 - DIM)))
    rows = _gather_kernel(emb_pad, idx.reshape(b * h * w))

    zqt, loss = pl.pallas_call(
        _epilogue_kernel,
        out_shape=(
            jax.ShapeDtypeStruct((b, c, h * w), jnp.float32),
            jax.ShapeDtypeStruct((1, 1), jnp.float32),
        ),
    )(rows.reshape(b, h * w, 128), zn)

    z_q_out = zqt.reshape(b, c, h, w)
    min_encoding_indices = idx.reshape(b, h, w)
    return (z_q_out, min_encoding_indices, loss.reshape(()))
